# baseline (speedup 1.0000x reference)
import functools

import jax
import jax.numpy as jnp
from jax import lax
from jax.experimental import pallas as pl
from jax.experimental.pallas import tpu as pltpu
from jax.experimental.pallas import tpu_sc as plsc

_GATHER_CHUNK = 128


@functools.lru_cache(maxsize=None)
def _make_sc_gather(V, D, B):
  info = plsc.get_sparse_core_info()
  NC, NS = info.num_cores, info.num_subcores
  NW = NC * NS
  assert B % NW == 0
  b_per_w = B // NW
  n_chunks = max(1, b_per_w // _GATHER_CHUNK)
  ch = b_per_w // n_chunks
  mesh = plsc.VectorSubcoreMesh(core_axis_name="c", subcore_axis_name="s")

  @functools.partial(
      pl.kernel,
      mesh=mesh,
      out_type=jax.ShapeDtypeStruct((B, D), jnp.float32),
      compiler_params=pltpu.CompilerParams(use_tc_tiling_on_sc=False),
      scratch_types=[
          pltpu.VMEM((b_per_w,), jnp.int32),
          pltpu.VMEM((b_per_w, D), jnp.float32),
          pltpu.SemaphoreType.DMA,
      ],
  )
  def gather(table_hbm, idx_hbm, out_hbm, idx_v, rows_v, sem):
    wid = lax.axis_index("s") * NC + lax.axis_index("c")
    base = wid * b_per_w
    pltpu.sync_copy(idx_hbm.at[pl.ds(base, b_per_w)], idx_v)
    copies = []
    for j in range(n_chunks):
      copies.append(
          pltpu.async_copy(
              table_hbm.at[idx_v.at[pl.ds(j * ch, ch)]],
              rows_v.at[pl.ds(j * ch, ch)],
              sem,
          ))
    for c in copies:
      c.wait()
    pltpu.sync_copy(rows_v, out_hbm.at[pl.ds(base, b_per_w)])

  return gather


def _proj_body(emb_ref, w_ref, b_ref, out_ref):
  out_ref[...] = (
      jnp.dot(emb_ref[...], w_ref[...], preferred_element_type=jnp.float32)
      + b_ref[...])


def _tc_proj(emb, W, b2d):
  R, D = emb.shape
  BB = 2048
  return pl.pallas_call(
      _proj_body,
      grid=(R // BB,),
      in_specs=[
          pl.BlockSpec((BB, D), lambda i: (i, 0)),
          pl.BlockSpec((D, D), lambda i: (0, 0)),
          pl.BlockSpec((1, D), lambda i: (0, 0)),
      ],
      out_specs=pl.BlockSpec((BB, D), lambda i: (i, 0)),
      out_shape=jax.ShapeDtypeStruct((R, D), jnp.float32),
  )(emb, W, b2d)


def kernel(ids, table, W, b):
  B = ids.shape[0]
  V, D = table.shape
  idx = ids.reshape(B)
  emb = _make_sc_gather(V, D, B)(table, idx)
  out = _tc_proj(emb, W, b.reshape(1, D))
  return out


# speedup vs baseline: 2.3021x; 2.3021x over previous
import functools

import jax
import jax.numpy as jnp
from jax import lax
from jax.experimental import pallas as pl
from jax.experimental.pallas import tpu as pltpu
from jax.experimental.pallas import tpu_sc as plsc

_S = 8
_CHUNK = 128


@functools.lru_cache(maxsize=None)
def _make_sc_tile_gather(T, D, B):
  info = plsc.get_sparse_core_info()
  NC, NS = info.num_cores, info.num_subcores
  NW = NC * NS
  assert B % NW == 0
  b_per_w = B // NW
  n_chunks = max(1, b_per_w // _CHUNK)
  ch = b_per_w // n_chunks
  mesh = plsc.VectorSubcoreMesh(core_axis_name="c", subcore_axis_name="s")

  @functools.partial(
      pl.kernel,
      mesh=mesh,
      out_type=jax.ShapeDtypeStruct((B, D), jnp.float32),
      scratch_types=[
          pltpu.VMEM((b_per_w,), jnp.int32),
          pltpu.VMEM((b_per_w,), jnp.int32),
          pltpu.VMEM((b_per_w, D), jnp.float32),
          pltpu.SemaphoreType.DMA,
      ],
  )
  def gather(tiles_hbm, tid_hbm, sub_hbm, out_hbm, tid_v, sub_v, rows_v, sem):
    wid = lax.axis_index("s") * NC + lax.axis_index("c")
    base = wid * b_per_w
    pltpu.sync_copy(tid_hbm.at[pl.ds(base, b_per_w)], tid_v)
    pltpu.sync_copy(sub_hbm.at[pl.ds(base, b_per_w)], sub_v)

    def grp(g, _):
      t16 = tid_v[pl.ds(g * 16, 16)]
      s16 = sub_v[pl.ds(g * 16, 16)]
      cps = []
      for k in range(16):
        cps.append(
            pltpu.async_copy(
                tiles_hbm.at[t16[k], s16[k]], rows_v.at[g * 16 + k], sem))
      for c in cps:
        c.wait()
      return 0

    lax.fori_loop(0, b_per_w // 16, grp, 0)
    pltpu.sync_copy(rows_v, out_hbm.at[pl.ds(base, b_per_w)])

  return gather


def _proj_body(emb_ref, w_ref, b_ref, out_ref):
  out_ref[...] = (
      jnp.dot(emb_ref[...], w_ref[...], preferred_element_type=jnp.float32)
      + b_ref[...])


def _tc_proj(emb, W, b2d):
  R, D = emb.shape
  BB = 2048
  return pl.pallas_call(
      _proj_body,
      grid=(R // BB,),
      in_specs=[
          pl.BlockSpec((BB, D), lambda i: (i, 0)),
          pl.BlockSpec((D, D), lambda i: (0, 0)),
          pl.BlockSpec((1, D), lambda i: (0, 0)),
      ],
      out_specs=pl.BlockSpec((BB, D), lambda i: (i, 0)),
      out_shape=jax.ShapeDtypeStruct((R, D), jnp.float32),
  )(emb, W, b2d)


def kernel(ids, table, W, b):
  B = ids.shape[0]
  V, D = table.shape
  idx = ids.reshape(B)
  tid = idx // _S
  sub = idx - tid * _S
  tiles = table.reshape(V // _S, _S, D)
  emb = _make_sc_tile_gather(V // _S, D, B)(tiles, tid, sub)
  out = _tc_proj(emb, W, b.reshape(1, D))
  return out
